# baseline (speedup 1.0000x reference)
import functools

import jax
import jax.numpy as jnp
from jax import lax
from jax.experimental import pallas as pl
from jax.experimental.pallas import tpu as pltpu
from jax.experimental.pallas import tpu_sc as plsc

NUM_EMBEDDINGS = 100000
D = 16
B, C, H, W = 1024, 3, 32, 32
HW = H * W

NC, NS, L = 2, 16, 16
NW = NC * NS
B_PER_W = B // NW
CHUNK = 128
NCH = HW // CHUNK
NPAIR = B_PER_W // 2
NBUF = 4

_mesh = plsc.VectorSubcoreMesh(
    core_axis_name="c", subcore_axis_name="s", num_cores=NC, num_subcores=NS
)


@functools.partial(
    pl.kernel,
    out_type=jax.ShapeDtypeStruct((B, D * HW), jnp.float32),
    mesh=_mesh,
    compiler_params=pltpu.CompilerParams(
        needs_layout_passes=False, use_tc_tiling_on_sc=False
    ),
    scratch_types=[
        pltpu.VMEM((2, C, NCH, CHUNK), jnp.int32),
        pltpu.VMEM((NBUF, C, CHUNK, D), jnp.float32),
        pltpu.VMEM((2, D * HW), jnp.float32),
        pltpu.SemaphoreType.DMA((NBUF,)),
        pltpu.SemaphoreType.DMA,
        pltpu.SemaphoreType.DMA,
    ],
)
def _bow_embed(
    idx_hbm, table_hbm, out_hbm, idx_v, rows_v, out_v, sem_g, sem_idx, sem_out
):
    wid = lax.axis_index("s") * NC + lax.axis_index("c")
    b0 = wid * B_PER_W
    col_base = lax.iota(jnp.int32, L) * HW

    def issue_gathers(ip, ch, rp):
        for c in range(C):
            pltpu.async_copy(
                table_hbm.at[idx_v.at[ip, c, ch]],
                rows_v.at[rp, c],
                sem_g.at[rp],
            )

    def wait_gathers(rp):
        for c in range(C):
            pltpu.make_async_copy(
                table_hbm.at[idx_v.at[0, c, 0]],
                rows_v.at[rp, c],
                sem_g.at[rp],
            ).wait()

    def drain_out():
        pltpu.make_async_copy(out_hbm.at[0], out_v.at[0], sem_out).wait()

    def drain_idx():
        pltpu.make_async_copy(idx_hbm.at[0], idx_v.at[0], sem_idx).wait()

    def compute_chunk(rp, op, ch):
        base = col_base + ch * CHUNK

        @plsc.parallel_loop(0, CHUNK, unroll=16)
        def per_pos(i):
            r = (rows_v[rp, 0, i] + rows_v[rp, 1, i] + rows_v[rp, 2, i]) * (
                1.0 / 3.0
            )
            plsc.store_scatter(out_v.at[op], [base + i], r)

    pltpu.sync_copy(idx_hbm.at[b0], idx_v.at[0])
    for ch in range(NBUF - 1):
        issue_gathers(0, ch, ch)
    pltpu.async_copy(idx_hbm.at[b0 + 1], idx_v.at[1], sem_idx)

    def per_pair(k2, _):
        for kk in range(2):
            p = kk
            k = 2 * k2 + kk
            b = b0 + k

            @pl.when(k >= 2)
            def _():
                drain_out()

            for ch in range(NCH):
                ahead = ch + NBUF - 1
                if ahead < NCH:
                    issue_gathers(p, ahead, ahead % NBUF)
                else:
                    if ahead == NCH:
                        @pl.when(k < B_PER_W - 1)
                        def _():
                            drain_idx()

                    @pl.when(k < B_PER_W - 1)
                    def _():
                        issue_gathers(1 - p, ahead - NCH, ahead % NBUF)

                wait_gathers(ch % NBUF)
                if ch == NCH - 1:
                    @pl.when(k < B_PER_W - 2)
                    def _():
                        pltpu.async_copy(
                            idx_hbm.at[b + 2], idx_v.at[p], sem_idx
                        )

                compute_chunk(ch % NBUF, p, ch)

            pltpu.async_copy(out_v.at[p], out_hbm.at[b], sem_out)
        return 0

    lax.fori_loop(0, NPAIR, per_pair, 0)
    drain_out()
    drain_out()


def kernel(inputs, table):
    idx = inputs.reshape(B, C, NCH, CHUNK).astype(jnp.int32)
    out = _bow_embed(idx, table)
    return out.reshape(B, D, H, W)


# speedup vs baseline: 1.0649x; 1.0649x over previous
import functools

import jax
import jax.numpy as jnp
from jax import lax
from jax.experimental import pallas as pl
from jax.experimental.pallas import tpu as pltpu
from jax.experimental.pallas import tpu_sc as plsc

NUM_EMBEDDINGS = 100000
D = 16
B, C, H, W = 1024, 3, 32, 32
HW = H * W

NC, NS, L = 2, 16, 16
NW = NC * NS
B_PER_W = B // NW
CHUNK = 128
NCH = HW // CHUNK
NPAIR = B_PER_W // 2
NBUF = 4

_mesh = plsc.VectorSubcoreMesh(
    core_axis_name="c", subcore_axis_name="s", num_cores=NC, num_subcores=NS
)


@functools.partial(
    pl.kernel,
    out_type=jax.ShapeDtypeStruct((B, D * HW), jnp.float32),
    mesh=_mesh,
    compiler_params=pltpu.CompilerParams(
        needs_layout_passes=False, use_tc_tiling_on_sc=False
    ),
    scratch_types=[
        pltpu.VMEM((2, C, NCH, CHUNK), jnp.int32),
        pltpu.VMEM((NBUF, CHUNK, D), jnp.float32),
        pltpu.VMEM((2, D * HW), jnp.float32),
        pltpu.SemaphoreType.DMA((NBUF,)),
        pltpu.SemaphoreType.DMA((NBUF,)),
        pltpu.SemaphoreType.DMA,
        pltpu.SemaphoreType.DMA,
    ],
)
def _bow_embed(
    idx_hbm, table_hbm, out_hbm, idx_v, rows_v, out_v, sem_a, sem_b,
    sem_idx, sem_out
):
    wid = lax.axis_index("s") * NC + lax.axis_index("c")
    b0 = wid * B_PER_W
    col_base = lax.iota(jnp.int32, L) * HW

    def issue_first(ip, ch, rp):
        pltpu.async_copy(
            table_hbm.at[idx_v.at[ip, 0, ch]], rows_v.at[rp], sem_a.at[rp]
        )

    def wait_first(rp):
        pltpu.make_async_copy(
            table_hbm.at[idx_v.at[0, 0, 0]], rows_v.at[rp], sem_a.at[rp]
        ).wait()

    def issue_adds(ip, ch, rp):
        for c in range(1, C):
            pltpu.async_copy(
                table_hbm.at[idx_v.at[ip, c, ch]],
                rows_v.at[rp],
                sem_b.at[rp],
                add=True,
            )

    def wait_adds(rp):
        for c in range(1, C):
            pltpu.make_async_copy(
                table_hbm.at[idx_v.at[0, 0, 0]], rows_v.at[rp], sem_b.at[rp]
            ).wait()

    def drain_out():
        pltpu.make_async_copy(out_hbm.at[0], out_v.at[0], sem_out).wait()

    def drain_idx():
        pltpu.make_async_copy(idx_hbm.at[0], idx_v.at[0], sem_idx).wait()

    def compute_chunk(rp, op, ch):
        base = col_base + ch * CHUNK

        @plsc.parallel_loop(0, CHUNK, unroll=4)
        def per_pos(i):
            r = rows_v[rp, i] * (1.0 / 3.0)
            plsc.store_scatter(out_v.at[op], [base + i], r)

    pltpu.sync_copy(idx_hbm.at[b0], idx_v.at[0])
    for ch in range(NBUF - 1):
        issue_first(0, ch, ch)
    for ch in range(NBUF - 2):
        wait_first(ch)
        issue_adds(0, ch, ch)
    pltpu.async_copy(idx_hbm.at[b0 + 1], idx_v.at[1], sem_idx)

    def per_pair(k2, _):
        for kk in range(2):
            p = kk
            k = 2 * k2 + kk
            b = b0 + k

            @pl.when(k >= 2)
            def _():
                drain_out()

            for ch in range(NCH):
                a3 = ch + NBUF - 1
                if a3 < NCH:
                    issue_first(p, a3, a3 % NBUF)
                else:
                    if a3 == NCH:
                        @pl.when(k < B_PER_W - 1)
                        def _():
                            drain_idx()

                    @pl.when(k < B_PER_W - 1)
                    def _():
                        issue_first(1 - p, a3 - NCH, a3 % NBUF)

                a2 = ch + NBUF - 2
                if a2 < NCH:
                    wait_first(a2 % NBUF)
                    issue_adds(p, a2, a2 % NBUF)
                else:

                    @pl.when(k < B_PER_W - 1)
                    def _():
                        wait_first(a2 % NBUF)
                        issue_adds(1 - p, a2 - NCH, a2 % NBUF)

                wait_adds(ch % NBUF)
                if ch == NCH - 1:
                    @pl.when(k < B_PER_W - 2)
                    def _():
                        pltpu.async_copy(
                            idx_hbm.at[b + 2], idx_v.at[p], sem_idx
                        )

                compute_chunk(ch % NBUF, p, ch)

            pltpu.async_copy(out_v.at[p], out_hbm.at[b], sem_out)
        return 0

    lax.fori_loop(0, NPAIR, per_pair, 0)
    drain_out()
    drain_out()


def kernel(inputs, table):
    idx = inputs.reshape(B, C, NCH, CHUNK).astype(jnp.int32)
    out = _bow_embed(idx, table)
    return out.reshape(B, D, H, W)


# speedup vs baseline: 1.0650x; 1.0001x over previous
import functools

import jax
import jax.numpy as jnp
from jax import lax
from jax.experimental import pallas as pl
from jax.experimental.pallas import tpu as pltpu
from jax.experimental.pallas import tpu_sc as plsc

NUM_EMBEDDINGS = 100000
D = 16
B, C, H, W = 1024, 3, 32, 32
HW = H * W

NC, NS, L = 2, 16, 16
NW = NC * NS
B_PER_W = B // NW
CHUNK = 128
NCH = HW // CHUNK
NPAIR = B_PER_W // 2
NBUF = 4

_mesh = plsc.VectorSubcoreMesh(
    core_axis_name="c", subcore_axis_name="s", num_cores=NC, num_subcores=NS
)


@functools.partial(
    pl.kernel,
    out_type=jax.ShapeDtypeStruct((B, D * HW), jnp.float32),
    mesh=_mesh,
    compiler_params=pltpu.CompilerParams(
        needs_layout_passes=False, use_tc_tiling_on_sc=False
    ),
    scratch_types=[
        pltpu.VMEM((2, C, NCH, CHUNK), jnp.int32),
        pltpu.VMEM((NBUF, CHUNK, D), jnp.float32),
        pltpu.VMEM((2, D * HW), jnp.float32),
        pltpu.SemaphoreType.DMA((NBUF,)),
        pltpu.SemaphoreType.DMA((NBUF,)),
        pltpu.SemaphoreType.DMA,
        pltpu.SemaphoreType.DMA,
    ],
)
def _bow_embed(
    idx_hbm, table_hbm, out_hbm, idx_v, rows_v, out_v, sem_a, sem_b,
    sem_idx, sem_out
):
    wid = lax.axis_index("s") * NC + lax.axis_index("c")
    b0 = wid * B_PER_W
    col_base = lax.iota(jnp.int32, L) * HW

    def issue_first(ip, ch, rp):
        pltpu.async_copy(
            table_hbm.at[idx_v.at[ip, 0, ch]], rows_v.at[rp], sem_a.at[rp]
        )

    def wait_first(rp):
        pltpu.make_async_copy(
            table_hbm.at[idx_v.at[0, 0, 0]], rows_v.at[rp], sem_a.at[rp]
        ).wait()

    def issue_adds(ip, ch, rp):
        for c in range(1, C):
            pltpu.async_copy(
                table_hbm.at[idx_v.at[ip, c, ch]],
                rows_v.at[rp],
                sem_b.at[rp],
                add=True,
            )

    def wait_adds(rp):
        for c in range(1, C):
            pltpu.make_async_copy(
                table_hbm.at[idx_v.at[0, 0, 0]], rows_v.at[rp], sem_b.at[rp]
            ).wait()

    def drain_out():
        pltpu.make_async_copy(out_hbm.at[0], out_v.at[0], sem_out).wait()

    def drain_idx():
        pltpu.make_async_copy(idx_hbm.at[0], idx_v.at[0], sem_idx).wait()

    col_vecs = [col_base + g for g in range(8)]
    span = (L - 1) * HW + 8

    def compute_chunk(rp, op, ch):
        @plsc.parallel_loop(0, CHUNK // 8, unroll=2)
        def per_group(j):
            off = pl.multiple_of(ch * CHUNK + j * 8, 8)
            tgt = out_v.at[op, pl.ds(off, span)]
            for g in range(8):
                r = rows_v[rp, j * 8 + g] * (1.0 / 3.0)
                plsc.store_scatter(tgt, [col_vecs[g]], r)

    pltpu.sync_copy(idx_hbm.at[b0], idx_v.at[0])
    for ch in range(NBUF - 1):
        issue_first(0, ch, ch)
    for ch in range(NBUF - 2):
        wait_first(ch)
        issue_adds(0, ch, ch)
    pltpu.async_copy(idx_hbm.at[b0 + 1], idx_v.at[1], sem_idx)

    def per_pair(k2, _):
        for kk in range(2):
            p = kk
            k = 2 * k2 + kk
            b = b0 + k

            @pl.when(k >= 2)
            def _():
                drain_out()

            for ch in range(NCH):
                a3 = ch + NBUF - 1
                if a3 < NCH:
                    issue_first(p, a3, a3 % NBUF)
                else:
                    if a3 == NCH:
                        @pl.when(k < B_PER_W - 1)
                        def _():
                            drain_idx()

                    @pl.when(k < B_PER_W - 1)
                    def _():
                        issue_first(1 - p, a3 - NCH, a3 % NBUF)

                a2 = ch + NBUF - 2
                if a2 < NCH:
                    wait_first(a2 % NBUF)
                    issue_adds(p, a2, a2 % NBUF)
                else:

                    @pl.when(k < B_PER_W - 1)
                    def _():
                        wait_first(a2 % NBUF)
                        issue_adds(1 - p, a2 - NCH, a2 % NBUF)

                wait_adds(ch % NBUF)
                if ch == NCH - 1:
                    @pl.when(k < B_PER_W - 2)
                    def _():
                        pltpu.async_copy(
                            idx_hbm.at[b + 2], idx_v.at[p], sem_idx
                        )

                compute_chunk(ch % NBUF, p, ch)

            pltpu.async_copy(out_v.at[p], out_hbm.at[b], sem_out)
        return 0

    lax.fori_loop(0, NPAIR, per_pair, 0)
    drain_out()
    drain_out()


def kernel(inputs, table):
    idx = inputs.reshape(B, C, NCH, CHUNK).astype(jnp.int32)
    out = _bow_embed(idx, table)
    return out.reshape(B, D, H, W)


# speedup vs baseline: 1.0903x; 1.0237x over previous
import functools

import jax
import jax.numpy as jnp
from jax import lax
from jax.experimental import pallas as pl
from jax.experimental.pallas import tpu as pltpu
from jax.experimental.pallas import tpu_sc as plsc

NUM_EMBEDDINGS = 100000
D = 16
B, C, H, W = 1024, 3, 32, 32
HW = H * W

NC, NS, L = 2, 16, 16
NW = NC * NS
B_PER_W = B // NW
CHUNK = 128
NCH = HW // CHUNK
NPAIR = B_PER_W // 2
NBUF = 8
LA = 5
LB = 3

_mesh = plsc.VectorSubcoreMesh(
    core_axis_name="c", subcore_axis_name="s", num_cores=NC, num_subcores=NS
)


@functools.partial(
    pl.kernel,
    out_type=jax.ShapeDtypeStruct((B, D * HW), jnp.float32),
    mesh=_mesh,
    compiler_params=pltpu.CompilerParams(
        needs_layout_passes=False, use_tc_tiling_on_sc=False
    ),
    scratch_types=[
        pltpu.VMEM((2, C, NCH, CHUNK), jnp.int32),
        pltpu.VMEM((NBUF, CHUNK, D), jnp.float32),
        pltpu.VMEM((2, D * HW), jnp.float32),
        pltpu.SemaphoreType.DMA((NBUF,)),
        pltpu.SemaphoreType.DMA((NBUF,)),
        pltpu.SemaphoreType.DMA,
        pltpu.SemaphoreType.DMA,
    ],
)
def _bow_embed(
    idx_hbm, table_hbm, out_hbm, idx_v, rows_v, out_v, sem_a, sem_b,
    sem_idx, sem_out
):
    wid = lax.axis_index("s") * NC + lax.axis_index("c")
    b0 = wid * B_PER_W
    col_base = lax.iota(jnp.int32, L) * HW

    def issue_first(ip, ch, rp):
        pltpu.async_copy(
            table_hbm.at[idx_v.at[ip, 0, ch]], rows_v.at[rp], sem_a.at[rp]
        )

    def wait_first(rp):
        pltpu.make_async_copy(
            table_hbm.at[idx_v.at[0, 0, 0]], rows_v.at[rp], sem_a.at[rp]
        ).wait()

    def issue_adds(ip, ch, rp):
        for c in range(1, C):
            pltpu.async_copy(
                table_hbm.at[idx_v.at[ip, c, ch]],
                rows_v.at[rp],
                sem_b.at[rp],
                add=True,
            )

    def wait_adds(rp):
        for c in range(1, C):
            pltpu.make_async_copy(
                table_hbm.at[idx_v.at[0, 0, 0]], rows_v.at[rp], sem_b.at[rp]
            ).wait()

    def drain_out():
        pltpu.make_async_copy(out_hbm.at[0], out_v.at[0], sem_out).wait()

    def drain_idx():
        pltpu.make_async_copy(idx_hbm.at[0], idx_v.at[0], sem_idx).wait()

    col_vecs = [col_base + g for g in range(8)]
    span = (L - 1) * HW + 8

    def compute_chunk(rp, op, ch):
        @plsc.parallel_loop(0, CHUNK // 8, unroll=2)
        def per_group(j):
            off = pl.multiple_of(ch * CHUNK + j * 8, 8)
            tgt = out_v.at[op, pl.ds(off, span)]
            for g in range(8):
                r = rows_v[rp, j * 8 + g] * (1.0 / 3.0)
                plsc.store_scatter(tgt, [col_vecs[g]], r)

    pltpu.sync_copy(idx_hbm.at[b0], idx_v.at[0])
    for ch in range(LA):
        issue_first(0, ch, ch)
    for ch in range(LB):
        wait_first(ch)
        issue_adds(0, ch, ch)
    pltpu.async_copy(idx_hbm.at[b0 + 1], idx_v.at[1], sem_idx)

    def per_pair(k2, _):
        for kk in range(2):
            p = kk
            k = 2 * k2 + kk
            b = b0 + k

            @pl.when(k >= 2)
            def _():
                drain_out()

            for ch in range(NCH):
                a3 = ch + LA
                if a3 < NCH:
                    issue_first(p, a3, a3 % NBUF)
                else:
                    if a3 == NCH:
                        @pl.when(k < B_PER_W - 1)
                        def _():
                            drain_idx()

                    @pl.when(k < B_PER_W - 1)
                    def _():
                        issue_first(1 - p, a3 - NCH, a3 % NBUF)

                a2 = ch + LB
                if a2 < NCH:
                    wait_first(a2 % NBUF)
                    issue_adds(p, a2, a2 % NBUF)
                else:

                    @pl.when(k < B_PER_W - 1)
                    def _():
                        wait_first(a2 % NBUF)
                        issue_adds(1 - p, a2 - NCH, a2 % NBUF)

                wait_adds(ch % NBUF)
                if ch == NCH - 1:
                    @pl.when(k < B_PER_W - 2)
                    def _():
                        pltpu.async_copy(
                            idx_hbm.at[b + 2], idx_v.at[p], sem_idx
                        )

                compute_chunk(ch % NBUF, p, ch)

            pltpu.async_copy(out_v.at[p], out_hbm.at[b], sem_out)
        return 0

    lax.fori_loop(0, NPAIR, per_pair, 0)
    drain_out()
    drain_out()


def kernel(inputs, table):
    idx = inputs.reshape(B, C, NCH, CHUNK).astype(jnp.int32)
    out = _bow_embed(idx, table)
    return out.reshape(B, D, H, W)
